# baseline (speedup 1.0000x reference)
import functools

import jax
import jax.numpy as jnp
from jax import lax
from jax.experimental import pallas as pl
from jax.experimental.pallas import tpu as pltpu
from jax.experimental.pallas import tpu_sc as plsc

EMB_DIM = 64
CHUNK = 128
K = 2


@functools.lru_cache(maxsize=None)
def _make_kernel(R: int, C: int):
    B = R * C
    info = plsc.get_sparse_core_info()
    NC, NS = info.num_cores, info.num_subcores
    NW = NC * NS
    n_chunks = B // CHUNK
    assert n_chunks * CHUNK == B and n_chunks % (NW * 2 * K) == 0
    assert R % CHUNK == 0
    bpt = R // CHUNK
    chunks_per_w = n_chunks // NW
    n_pairs = chunks_per_w // (2 * K)

    mesh = plsc.VectorSubcoreMesh(core_axis_name="c", subcore_axis_name="s")

    @functools.partial(
        pl.kernel,
        out_type=jax.ShapeDtypeStruct(
            (C, EMB_DIM // 8, R // 128, 8, 128), jnp.float32
        ),
        mesh=mesh,
        scratch_types=[
            pltpu.VMEM((chunks_per_w, CHUNK), jnp.int32),
            pltpu.VMEM((2 * K, CHUNK, EMB_DIM), jnp.float32),
            pltpu.VMEM((2 * K, EMB_DIM // 8, 8, CHUNK), jnp.float32),
            pltpu.SemaphoreType.DMA,
            pltpu.SemaphoreType.DMA,
            pltpu.SemaphoreType.DMA,
            pltpu.SemaphoreType.DMA,
        ],
        compiler_params=pltpu.CompilerParams(
            use_tc_tiling_on_sc=False, needs_layout_passes=False
        ),
    )
    def k(idx_hbm, table_hbm, out_hbm, idx_v, rows_v, tr_v, gsA, gsB, ssA, ssB):
        wid = lax.axis_index("s") * NC + lax.axis_index("c")
        base_chunk = wid * chunks_per_w
        pltpu.sync_copy(idx_hbm.at[pl.ds(base_chunk, chunks_per_w)], idx_v)
        iota16 = lax.iota(jnp.int32, 16)

        def fire_gathers(half, j0, sem):
            for b in range(K):
                pltpu.async_copy(
                    table_hbm.at[idx_v.at[j0 + b]], rows_v.at[half * K + b], sem
                )

        def drain_gathers(half, sem):
            for b in range(K):
                pltpu.make_async_copy(
                    table_hbm.at[idx_v.at[0]], rows_v.at[half * K + b], sem
                ).wait()

        def transpose_chunks(half):
            for b in range(K):
                rb = rows_v.at[half * K + b]
                tb = tr_v.at[half * K + b]

                def tbody(i, carry):
                    l0 = i * 16
                    lidx = iota16 + l0
                    for c in range(EMB_DIM):
                        vals = plsc.load_gather(
                            rb, [lidx, jnp.full((16,), c, jnp.int32)]
                        )
                        tb[c // 8, c % 8, pl.ds(l0, 16)] = vals
                    return carry

                lax.fori_loop(0, CHUNK // 16, tbody, 0)

        def fire_scatters(half, j0, sem):
            for b in range(K):
                g = base_chunk + j0 + b
                t = g // bpt
                bb = g % bpt
                for cc in range(EMB_DIM // 8):
                    pltpu.async_copy(
                        tr_v.at[half * K + b, cc],
                        out_hbm.at[t, cc, bb],
                        sem,
                    )

        def drain_scatters(half, sem):
            for b in range(K):
                for cc in range(EMB_DIM // 8):
                    pltpu.make_async_copy(
                        tr_v.at[half * K + b, cc],
                        out_hbm.at[0, cc, 0],
                        sem,
                    ).wait()

        fire_gathers(0, 0, gsA)

        def body(p, carry):
            jA = p * 2 * K
            jB = jA + K

            @pl.when(p > 0)
            def _():
                drain_scatters(1, ssB)

            fire_gathers(1, jB, gsB)
            drain_gathers(0, gsA)

            @pl.when(p > 0)
            def _():
                drain_scatters(0, ssA)

            transpose_chunks(0)
            fire_scatters(0, jA, ssA)

            @pl.when(p < n_pairs - 1)
            def _():
                fire_gathers(0, jA + 2 * K, gsA)

            drain_gathers(1, gsB)
            transpose_chunks(1)
            fire_scatters(1, jB, ssB)
            return carry

        lax.fori_loop(0, n_pairs, body, 0)
        drain_scatters(0, ssA)
        drain_scatters(1, ssB)

    return k


def kernel(token_ids, E):
    R, C = token_ids.shape
    B = R * C
    idx2d = token_ids.T.reshape(B // CHUNK, CHUNK).astype(jnp.int32)
    out5 = _make_kernel(R, C)(idx2d, E)
    return out5.transpose(2, 4, 0, 1, 3).reshape(R, C, EMB_DIM)


# speedup vs baseline: 1.5508x; 1.5508x over previous
import functools

import jax
import jax.numpy as jnp
from jax import lax
from jax.experimental import pallas as pl
from jax.experimental.pallas import tpu as pltpu
from jax.experimental.pallas import tpu_sc as plsc

EMB_DIM = 64
CHUNK = 128
K = 5


@functools.lru_cache(maxsize=None)
def _make_kernel(B: int):
    info = plsc.get_sparse_core_info()
    NC, NS = info.num_cores, info.num_subcores
    NW = NC * NS
    n_chunks = B // CHUNK
    assert n_chunks * CHUNK == B and n_chunks % (NW * 2 * K) == 0
    chunks_per_w = n_chunks // NW
    n_pairs = chunks_per_w // (2 * K)

    mesh = plsc.VectorSubcoreMesh(core_axis_name="c", subcore_axis_name="s")

    @functools.partial(
        pl.kernel,
        out_type=jax.ShapeDtypeStruct((B, EMB_DIM), jnp.float32),
        mesh=mesh,
        scratch_types=[
            pltpu.VMEM((chunks_per_w, CHUNK), jnp.int32),
            pltpu.VMEM((2 * K, CHUNK, EMB_DIM), jnp.float32),
            pltpu.SemaphoreType.DMA,
            pltpu.SemaphoreType.DMA,
            pltpu.SemaphoreType.DMA,
            pltpu.SemaphoreType.DMA,
        ],
        compiler_params=pltpu.CompilerParams(use_tc_tiling_on_sc=False),
    )
    def k(idx_hbm, table_hbm, out_hbm, idx_v, rows_v, gsA, gsB, ssA, ssB):
        wid = lax.axis_index("s") * NC + lax.axis_index("c")
        base_chunk = wid * chunks_per_w
        pltpu.sync_copy(idx_hbm.at[pl.ds(base_chunk, chunks_per_w)], idx_v)

        def fire_gathers(half, j0, sem):
            for b in range(K):
                pltpu.async_copy(
                    table_hbm.at[idx_v.at[j0 + b]], rows_v.at[half * K + b], sem
                )

        def drain_gathers(half, sem):
            for b in range(K):
                pltpu.make_async_copy(
                    table_hbm.at[idx_v.at[0]], rows_v.at[half * K + b], sem
                ).wait()

        def fire_scatters(half, j0, sem):
            for b in range(K):
                pltpu.async_copy(
                    rows_v.at[half * K + b],
                    out_hbm.at[pl.ds((base_chunk + j0 + b) * CHUNK, CHUNK)],
                    sem,
                )

        def drain_scatters(half, sem):
            for b in range(K):
                pltpu.make_async_copy(
                    rows_v.at[half * K + b],
                    out_hbm.at[pl.ds(0, CHUNK)],
                    sem,
                ).wait()

        fire_gathers(0, 0, gsA)

        def body(p, carry):
            jA = p * 2 * K
            jB = jA + K

            @pl.when(p > 0)
            def _():
                drain_scatters(1, ssB)

            fire_gathers(1, jB, gsB)
            drain_gathers(0, gsA)
            fire_scatters(0, jA, ssA)

            @pl.when(p < n_pairs - 1)
            def _():
                drain_scatters(0, ssA)
                fire_gathers(0, jA + 2 * K, gsA)

            drain_gathers(1, gsB)
            fire_scatters(1, jB, ssB)
            return carry

        lax.fori_loop(0, n_pairs, body, 0)
        drain_scatters(0, ssA)
        drain_scatters(1, ssB)

    return k


def kernel(token_ids, E):
    B = token_ids.size
    R, C = token_ids.shape
    idx2d = token_ids.T.reshape(B // CHUNK, CHUNK).astype(jnp.int32)
    out = _make_kernel(B)(idx2d, E)
    return out.reshape(C, R, EMB_DIM).transpose(1, 0, 2)
